# Initial kernel scaffold; baseline (speedup 1.0000x reference)
import functools

import jax
import jax.numpy as jnp
from jax import lax
from jax.experimental import pallas as pl
from jax.experimental.pallas import tpu as pltpu
from jax.experimental.pallas import tpu_sc as plsc

NN = 10000
NE = 320000
D = 128
NG = 64
NC, NS, L = 2, 16, 16
NW = NC * NS
EPT = NE // NW
K = 80
NCH = EPT // K
RPT = NN // NS
ZR = 125

RB = 1000
GRID = NN // RB

_mesh = plsc.VectorSubcoreMesh(core_axis_name="c", subcore_axis_name="s")


@functools.partial(
    pl.kernel,
    out_type=jax.ShapeDtypeStruct((NC, NN, L), jnp.float32),
    mesh=_mesh,
    scratch_types=[
        pltpu.VMEM((NCH, K), jnp.int32),
        pltpu.VMEM((K, L), jnp.float32),
        pltpu.VMEM((ZR, L), jnp.float32),
        pltpu.VMEM_SHARED((NN, L), jnp.float32),
    ],
)
def _sc_deg(dst3, out, idx_v, ones_v, zbuf_v, deg_sh):
    c = lax.axis_index("c")
    s = lax.axis_index("s")
    wid = s * NC + c
    pltpu.sync_copy(dst3.at[wid], idx_v)

    def fill_ones(i, carry):
        ones_v[i, :] = jnp.ones((L,), jnp.float32)
        return carry

    lax.fori_loop(0, K, fill_ones, 0)

    def fill_zero(i, carry):
        zbuf_v[i, :] = jnp.zeros((L,), jnp.float32)
        return carry

    lax.fori_loop(0, ZR, fill_zero, 0)
    for r in range(RPT // ZR):
        pltpu.sync_copy(zbuf_v, deg_sh.at[pl.ds(s * RPT + r * ZR, ZR)])
    plsc.subcore_barrier()

    def body(j, carry):
        pltpu.sync_copy(ones_v, deg_sh.at[idx_v.at[j]], add=True)
        return carry

    lax.fori_loop(0, NCH, body, 0)
    plsc.subcore_barrier()
    for r in range(RPT // ZR):
        rows = pl.ds(s * RPT + r * ZR, ZR)
        pltpu.sync_copy(deg_sh.at[rows], zbuf_v)
        pltpu.sync_copy(zbuf_v, out.at[c, rows])


@functools.partial(
    pl.kernel,
    out_type=jax.ShapeDtypeStruct((NC, NN, D), jnp.float32),
    mesh=_mesh,
    scratch_types=[
        pltpu.VMEM((NCH, K), jnp.int32),
        pltpu.VMEM((NCH, K), jnp.int32),
        pltpu.VMEM((K, D), jnp.float32),
        pltpu.VMEM((ZR, D), jnp.float32),
        pltpu.VMEM_SHARED((NN, D), jnp.float32),
        pltpu.SemaphoreType.DMA,
    ],
)
def _sc_scatter(src3, dst3, g_hbm, out, isrc_v, idst_v, rows_v, zbuf_v,
                acc_sh, sem):
    c = lax.axis_index("c")
    s = lax.axis_index("s")
    wid = s * NC + c
    pltpu.sync_copy(src3.at[wid], isrc_v)
    pltpu.sync_copy(dst3.at[wid], idst_v)

    def fill_zero(i, carry):
        for j in range(D // L):
            zbuf_v[i, pl.ds(j * L, L)] = jnp.zeros((L,), jnp.float32)
        return carry

    lax.fori_loop(0, ZR, fill_zero, 0)
    for r in range(RPT // ZR):
        pltpu.sync_copy(zbuf_v, acc_sh.at[pl.ds(s * RPT + r * ZR, ZR)])
    plsc.subcore_barrier()

    def body(j, carry):
        pltpu.async_copy(g_hbm.at[isrc_v.at[j]], rows_v, sem).wait()
        pltpu.sync_copy(rows_v, acc_sh.at[idst_v.at[j]], add=True)
        return carry

    lax.fori_loop(0, NCH, body, 0)
    plsc.subcore_barrier()
    for r in range(RPT // ZR):
        rows = pl.ds(s * RPT + r * ZR, ZR)
        pltpu.sync_copy(acc_sh.at[rows], zbuf_v)
        pltpu.sync_copy(zbuf_v, out.at[c, rows])


def _dinv(deg_ref):
    d = deg_ref[0, :, 0] + deg_ref[1, :, 0] + 1.0
    return lax.rsqrt(d)[:, None]


def _tc1_body(x_ref, w_ref, deg_ref, o_ref):
    h = jnp.dot(x_ref[...], w_ref[...], preferred_element_type=jnp.float32)
    o_ref[...] = h * _dinv(deg_ref)


_tc1 = pl.pallas_call(
    _tc1_body,
    grid=(GRID,),
    in_specs=[
        pl.BlockSpec((RB, D), lambda i: (i, 0)),
        pl.BlockSpec((D, D), lambda i: (0, 0)),
        pl.BlockSpec((NC, RB, L), lambda i: (0, i, 0)),
    ],
    out_specs=pl.BlockSpec((RB, D), lambda i: (i, 0)),
    out_shape=jax.ShapeDtypeStruct((NN, D), jnp.float32),
)


def _tc2_body(s_ref, g_ref, deg_ref, b_ref, w_ref, o_ref):
    dinv = _dinv(deg_ref)
    h = jnp.maximum((s_ref[0] + s_ref[1] + g_ref[...]) * dinv + b_ref[...],
                    0.0)
    o_ref[...] = jnp.dot(h, w_ref[...],
                         preferred_element_type=jnp.float32) * dinv


_tc2 = pl.pallas_call(
    _tc2_body,
    grid=(GRID,),
    in_specs=[
        pl.BlockSpec((NC, RB, D), lambda i: (0, i, 0)),
        pl.BlockSpec((RB, D), lambda i: (i, 0)),
        pl.BlockSpec((NC, RB, L), lambda i: (0, i, 0)),
        pl.BlockSpec((1, D), lambda i: (0, 0)),
        pl.BlockSpec((D, D), lambda i: (0, 0)),
    ],
    out_specs=pl.BlockSpec((RB, D), lambda i: (i, 0)),
    out_shape=jax.ShapeDtypeStruct((NN, D), jnp.float32),
)


def _tc3_body(s_ref, g_ref, deg_ref, b_ref, batch_ref, w_ref, bo_ref, o_ref,
              sums_ref, cnt_ref):
    i = pl.program_id(0)

    @pl.when(i == 0)
    def _():
        sums_ref[...] = jnp.zeros_like(sums_ref)
        cnt_ref[...] = jnp.zeros_like(cnt_ref)

    dinv = _dinv(deg_ref)
    h = jnp.maximum((s_ref[0] + s_ref[1] + g_ref[...]) * dinv + b_ref[...],
                    0.0)
    bt = batch_ref[0, 0, :]
    oh = (lax.broadcasted_iota(jnp.int32, (NG, RB), 0) == bt[None, :]).astype(
        jnp.float32)
    sums_ref[...] += jnp.dot(oh, h, preferred_element_type=jnp.float32)
    cnt_ref[...] += jnp.broadcast_to(
        jnp.sum(oh, axis=1, keepdims=True), (NG, D))

    @pl.when(i == GRID - 1)
    def _():
        pooled = sums_ref[...] / jnp.maximum(cnt_ref[...], 1.0)
        o_ref[...] = jnp.dot(pooled, w_ref[...],
                             preferred_element_type=jnp.float32) + bo_ref[...]


_tc3 = pl.pallas_call(
    _tc3_body,
    grid=(GRID,),
    in_specs=[
        pl.BlockSpec((NC, RB, D), lambda i: (0, i, 0)),
        pl.BlockSpec((RB, D), lambda i: (i, 0)),
        pl.BlockSpec((NC, RB, L), lambda i: (0, i, 0)),
        pl.BlockSpec((1, D), lambda i: (0, 0)),
        pl.BlockSpec((1, 1, RB), lambda i: (i, 0, 0)),
        pl.BlockSpec((D, D), lambda i: (0, 0)),
        pl.BlockSpec((1, D), lambda i: (0, 0)),
    ],
    out_specs=pl.BlockSpec((NG, D), lambda i: (0, 0)),
    out_shape=jax.ShapeDtypeStruct((NG, D), jnp.float32),
    scratch_shapes=[
        pltpu.VMEM((NG, D), jnp.float32),
        pltpu.VMEM((NG, D), jnp.float32),
    ],
)


def kernel(x, edge_index, batch, W_in, b_in, W_h, b_h, W_out, b_out):
    ei = edge_index.astype(jnp.int32)
    src3 = ei[0].reshape(NW, NCH, K)
    dst3 = ei[1].reshape(NW, NCH, K)
    batch3 = batch.astype(jnp.int32).reshape(GRID, 1, RB)

    deg2 = _sc_deg(dst3)
    g1 = _tc1(x, W_in, deg2)
    s1 = _sc_scatter(src3, dst3, g1)
    g2 = _tc2(s1, g1, deg2, b_in.reshape(1, D), W_h)
    s2 = _sc_scatter(src3, dst3, g2)
    return _tc3(s2, g2, deg2, b_h.reshape(1, D), batch3,
                W_out, b_out.reshape(1, D))


# speedup vs baseline: 7.8762x; 7.8762x over previous
import functools

import jax
import jax.numpy as jnp
from jax import lax
from jax.experimental import pallas as pl
from jax.experimental.pallas import tpu as pltpu
from jax.experimental.pallas import tpu_sc as plsc

NN = 10000
NE = 320000
D = 128
NG = 64
NC, NS, L = 2, 16, 16
NW = NC * NS
K = 80
NCHS = (NE // NS) // K

NH = NN // NC
SWRT = 312
SZR = 104
STAIL = NH - NS * SWRT
ACC_ROWS = 5008

ER = NE // D

RB = 1000
GRID = NN // RB

_mesh = plsc.VectorSubcoreMesh(core_axis_name="c", subcore_axis_name="s")


def _zero_init(zeros_hbm, zbuf_v, sh, s, zr, wrt, tail, tbase):
    pltpu.sync_copy(zeros_hbm, zbuf_v)
    for r in range(wrt // zr):
        pltpu.sync_copy(zbuf_v, sh.at[pl.ds(s * wrt + r * zr, zr)])

    @pl.when(s == 0)
    def _():
        pltpu.sync_copy(zbuf_v.at[pl.ds(0, tail)],
                        sh.at[pl.ds(tbase, tail)])


def _writeout(zbuf_v, sh, out, c, s, zr, wrt, tail, tbase):
    for r in range(wrt // zr):
        rows = pl.ds(s * wrt + r * zr, zr)
        pltpu.sync_copy(sh.at[rows], zbuf_v)
        pltpu.sync_copy(zbuf_v, out.at[c, rows])

    @pl.when(s == 0)
    def _():
        rows = pl.ds(tbase, tail)
        pltpu.sync_copy(sh.at[rows], zbuf_v.at[pl.ds(0, tail)])
        pltpu.sync_copy(zbuf_v.at[pl.ds(0, tail)], out.at[c, rows])


@functools.partial(
    pl.kernel,
    out_type=jax.ShapeDtypeStruct((NC, NH, D), jnp.float32),
    mesh=_mesh,
    scratch_types=[
        pltpu.VMEM((NCHS, K), jnp.int32),
        pltpu.VMEM((NCHS, K), jnp.int32),
        pltpu.VMEM((K, D), jnp.float32),
        pltpu.VMEM((SZR, D), jnp.float32),
        pltpu.VMEM_SHARED((ACC_ROWS, D), jnp.float32),
        pltpu.SemaphoreType.DMA,
    ],
)
def _sc_scatter(src2, ldst4, g_hbm, zeros_hbm, out, isrc_v, idst_v, rows_v,
                zbuf_v, acc_sh, sem):
    c = lax.axis_index("c")
    s = lax.axis_index("s")
    pltpu.sync_copy(src2.at[s], isrc_v)
    pltpu.sync_copy(ldst4.at[c, s], idst_v)
    _zero_init(zeros_hbm, zbuf_v, acc_sh, s, SZR, SWRT, STAIL, NS * SWRT)
    plsc.subcore_barrier()

    def body(j, carry):
        pltpu.async_copy(g_hbm.at[isrc_v.at[j]], rows_v, sem).wait()
        pltpu.sync_copy(rows_v, acc_sh.at[idst_v.at[j]], add=True)
        return carry

    lax.fori_loop(0, NCHS, body, 0)
    plsc.subcore_barrier()
    _writeout(zbuf_v, acc_sh, out, c, s, SZR, SWRT, STAIL, NS * SWRT)


def _loc_body(dst_ref, o_ref):
    d = dst_ref[...]
    for c in range(NC):
        t = d - c * NH
        o_ref[c] = jnp.where((t >= 0) & (t < NH), t, NH)


_tc_loc = pl.pallas_call(
    _loc_body,
    out_shape=jax.ShapeDtypeStruct((NC, ER, D), jnp.int32),
)


def _dinv(deg_ref):
    d = deg_ref[0, :, 0] + 1.0
    return lax.rsqrt(d)[:, None]


_HALF_SPEC = pl.BlockSpec((1, RB, D),
                          lambda i: (i // (NH // RB), i % (NH // RB), 0))


def _tc1_body(x_ref, w_ref, deg_ref, o_ref):
    h = jnp.dot(x_ref[...], w_ref[...], preferred_element_type=jnp.float32)
    o_ref[...] = h * _dinv(deg_ref)


_tc1 = pl.pallas_call(
    _tc1_body,
    grid=(GRID,),
    in_specs=[
        pl.BlockSpec((RB, D), lambda i: (i, 0)),
        pl.BlockSpec((D, D), lambda i: (0, 0)),
        _HALF_SPEC,
    ],
    out_specs=pl.BlockSpec((RB, D), lambda i: (i, 0)),
    out_shape=jax.ShapeDtypeStruct((NN, D), jnp.float32),
)


def _post_body(s_ref, g_ref, deg_ref, b_ref, o_ref):
    dinv = _dinv(deg_ref)
    t = (s_ref[0] + g_ref[...]) * dinv + b_ref[...]
    o_ref[...] = jnp.maximum(t, 0.0)


_tc_post = pl.pallas_call(
    _post_body,
    grid=(GRID,),
    in_specs=[
        _HALF_SPEC,
        pl.BlockSpec((RB, D), lambda i: (i, 0)),
        _HALF_SPEC,
        pl.BlockSpec((1, D), lambda i: (0, 0)),
    ],
    out_specs=pl.BlockSpec((RB, D), lambda i: (i, 0)),
    out_shape=jax.ShapeDtypeStruct((NN, D), jnp.float32),
)


def _tc3_body(h_ref, batch_ref, w_ref, bo_ref, o_ref, sums_ref, cnt_ref):
    i = pl.program_id(0)

    @pl.when(i == 0)
    def _():
        sums_ref[...] = jnp.zeros_like(sums_ref)
        cnt_ref[...] = jnp.zeros_like(cnt_ref)

    h = h_ref[...]
    bt = batch_ref[0, 0, :]
    oh = (lax.broadcasted_iota(jnp.int32, (NG, RB), 0) == bt[None, :]).astype(
        jnp.float32)
    sums_ref[...] += jnp.dot(oh, h, preferred_element_type=jnp.float32)
    cnt_ref[...] += jnp.broadcast_to(
        jnp.sum(oh, axis=1, keepdims=True), (NG, D))

    @pl.when(i == GRID - 1)
    def _():
        pooled = sums_ref[...] / jnp.maximum(cnt_ref[...], 1.0)
        o_ref[...] = jnp.dot(pooled, w_ref[...],
                             preferred_element_type=jnp.float32) + bo_ref[...]


_tc3 = pl.pallas_call(
    _tc3_body,
    grid=(GRID,),
    in_specs=[
        pl.BlockSpec((RB, D), lambda i: (i, 0)),
        pl.BlockSpec((1, 1, RB), lambda i: (i, 0, 0)),
        pl.BlockSpec((D, D), lambda i: (0, 0)),
        pl.BlockSpec((1, D), lambda i: (0, 0)),
    ],
    out_specs=pl.BlockSpec((NG, D), lambda i: (0, 0)),
    out_shape=jax.ShapeDtypeStruct((NG, D), jnp.float32),
    scratch_shapes=[
        pltpu.VMEM((NG, D), jnp.float32),
        pltpu.VMEM((NG, D), jnp.float32),
    ],
)


def kernel(x, edge_index, batch, W_in, b_in, W_h, b_h, W_out, b_out):
    ei = edge_index.astype(jnp.int32)
    src2 = ei[0].reshape(NS, NCHS, K)
    batch3 = batch.astype(jnp.int32).reshape(GRID, 1, RB)

    ldst4 = _tc_loc(ei[1].reshape(ER, D)).reshape(NC, NS, NCHS, K)

    zerosD = jnp.zeros((SZR, D), jnp.float32)
    onesT = jnp.ones((NN, D), jnp.float32)

    deg2 = _sc_scatter(src2, ldst4, onesT, zerosD)
    Ws = jnp.stack([W_in, W_h])
    bs = jnp.stack([b_in.reshape(1, D), b_h.reshape(1, D)])

    def step(h, wb):
        W, b = wb
        g = _tc1(h, W, deg2)
        s = _sc_scatter(src2, ldst4, g, zerosD)
        return _tc_post(s, g, deg2, b), None

    h2, _ = lax.scan(step, x, (Ws, bs))
    return _tc3(h2, batch3, W_out, b_out.reshape(1, D))


# speedup vs baseline: 9.2588x; 1.1755x over previous
import functools

import jax
import jax.numpy as jnp
from jax import lax
from jax.experimental import pallas as pl
from jax.experimental.pallas import tpu as pltpu
from jax.experimental.pallas import tpu_sc as plsc

NN = 10000
NE = 320000
D = 128
NG = 64
NC, NS, L = 2, 16, 16
NW = NC * NS
K = 80
NCHS = (NE // NS) // K

NH = NN // NC
SWRT = 312
SZR = 104
STAIL = NH - NS * SWRT
ACC_ROWS = 5008

ER = NE // D

RB = 1000
GRID = NN // RB

_mesh = plsc.VectorSubcoreMesh(core_axis_name="c", subcore_axis_name="s")


def _zero_init(zeros_hbm, zbuf_v, sh, s, zr, wrt, tail, tbase):
    pltpu.sync_copy(zeros_hbm, zbuf_v)
    for r in range(wrt // zr):
        pltpu.sync_copy(zbuf_v, sh.at[pl.ds(s * wrt + r * zr, zr)])

    @pl.when(s == 0)
    def _():
        pltpu.sync_copy(zbuf_v.at[pl.ds(0, tail)],
                        sh.at[pl.ds(tbase, tail)])


def _writeout(zbuf_v, sh, out, c, s, zr, wrt, tail, tbase):
    for r in range(wrt // zr):
        rows = pl.ds(s * wrt + r * zr, zr)
        pltpu.sync_copy(sh.at[rows], zbuf_v)
        pltpu.sync_copy(zbuf_v, out.at[c, rows])

    @pl.when(s == 0)
    def _():
        rows = pl.ds(tbase, tail)
        pltpu.sync_copy(sh.at[rows], zbuf_v.at[pl.ds(0, tail)])
        pltpu.sync_copy(zbuf_v.at[pl.ds(0, tail)], out.at[c, rows])


@functools.partial(
    pl.kernel,
    out_type=jax.ShapeDtypeStruct((NC, NH, D), jnp.float32),
    mesh=_mesh,
    scratch_types=[
        pltpu.VMEM((NCHS, K), jnp.int32),
        pltpu.VMEM((NCHS, K), jnp.int32),
        pltpu.VMEM((K, D), jnp.float32),
        pltpu.VMEM((SZR, D), jnp.float32),
        pltpu.VMEM_SHARED((ACC_ROWS, D), jnp.float32),
        pltpu.SemaphoreType.DMA,
    ],
)
def _sc_scatter(src2, ldst4, g_hbm, zeros_hbm, out, isrc_v, idst_v, rows_v,
                zbuf_v, acc_sh, sem):
    c = lax.axis_index("c")
    s = lax.axis_index("s")
    pltpu.sync_copy(src2.at[s], isrc_v)
    pltpu.sync_copy(ldst4.at[c, s], idst_v)
    _zero_init(zeros_hbm, zbuf_v, acc_sh, s, SZR, SWRT, STAIL, NS * SWRT)
    plsc.subcore_barrier()

    def body(j, carry):
        pltpu.async_copy(g_hbm.at[isrc_v.at[j]], rows_v, sem).wait()
        pltpu.sync_copy(rows_v, acc_sh.at[idst_v.at[j]], add=True)
        return carry

    lax.fori_loop(0, NCHS, body, 0)
    plsc.subcore_barrier()
    _writeout(zbuf_v, acc_sh, out, c, s, SZR, SWRT, STAIL, NS * SWRT)


@functools.partial(
    pl.kernel,
    out_type=jax.ShapeDtypeStruct((NC, NH, D), jnp.float32),
    mesh=_mesh,
    scratch_types=[
        pltpu.VMEM((NCHS, K), jnp.int32),
        pltpu.VMEM((K, D), jnp.float32),
        pltpu.VMEM((SZR, D), jnp.float32),
        pltpu.VMEM_SHARED((ACC_ROWS, D), jnp.float32),
    ],
)
def _sc_deg(ldst4, ones_hbm, zeros_hbm, out, idst_v, rows_v, zbuf_v, acc_sh):
    c = lax.axis_index("c")
    s = lax.axis_index("s")
    pltpu.sync_copy(ldst4.at[c, s], idst_v)
    pltpu.sync_copy(ones_hbm, rows_v)
    _zero_init(zeros_hbm, zbuf_v, acc_sh, s, SZR, SWRT, STAIL, NS * SWRT)
    plsc.subcore_barrier()

    def body(j, carry):
        pltpu.sync_copy(rows_v, acc_sh.at[idst_v.at[j]], add=True)
        return carry

    lax.fori_loop(0, NCHS, body, 0)
    plsc.subcore_barrier()
    _writeout(zbuf_v, acc_sh, out, c, s, SZR, SWRT, STAIL, NS * SWRT)


def _loc_body(dst_ref, o_ref):
    d = dst_ref[...]
    for c in range(NC):
        t = d - c * NH
        o_ref[c] = jnp.where((t >= 0) & (t < NH), t, NH)


_tc_loc = pl.pallas_call(
    _loc_body,
    out_shape=jax.ShapeDtypeStruct((NC, ER, D), jnp.int32),
)


def _dinv(deg_ref):
    d = deg_ref[0, :, 0] + 1.0
    return lax.rsqrt(d)[:, None]


_HALF_SPEC = pl.BlockSpec((1, RB, D),
                          lambda i: (i // (NH // RB), i % (NH // RB), 0))


def _tc1_body(x_ref, w_ref, deg_ref, o_ref):
    h = jnp.dot(x_ref[...], w_ref[...], preferred_element_type=jnp.float32)
    o_ref[...] = h * _dinv(deg_ref)


_tc1 = pl.pallas_call(
    _tc1_body,
    grid=(GRID,),
    in_specs=[
        pl.BlockSpec((RB, D), lambda i: (i, 0)),
        pl.BlockSpec((D, D), lambda i: (0, 0)),
        _HALF_SPEC,
    ],
    out_specs=pl.BlockSpec((RB, D), lambda i: (i, 0)),
    out_shape=jax.ShapeDtypeStruct((NN, D), jnp.float32),
)


def _post_body(s_ref, g_ref, deg_ref, b_ref, o_ref):
    dinv = _dinv(deg_ref)
    t = (s_ref[0] + g_ref[...]) * dinv + b_ref[...]
    o_ref[...] = jnp.maximum(t, 0.0)


_tc_post = pl.pallas_call(
    _post_body,
    grid=(GRID,),
    in_specs=[
        _HALF_SPEC,
        pl.BlockSpec((RB, D), lambda i: (i, 0)),
        _HALF_SPEC,
        pl.BlockSpec((1, D), lambda i: (0, 0)),
    ],
    out_specs=pl.BlockSpec((RB, D), lambda i: (i, 0)),
    out_shape=jax.ShapeDtypeStruct((NN, D), jnp.float32),
)


def _tc3_body(h_ref, batch_ref, w_ref, bo_ref, o_ref, sums_ref, cnt_ref):
    i = pl.program_id(0)

    @pl.when(i == 0)
    def _():
        sums_ref[...] = jnp.zeros_like(sums_ref)
        cnt_ref[...] = jnp.zeros_like(cnt_ref)

    h = h_ref[...]
    bt = batch_ref[0, 0, :]
    oh = (lax.broadcasted_iota(jnp.int32, (NG, RB), 0) == bt[None, :]).astype(
        jnp.float32)
    sums_ref[...] += jnp.dot(oh, h, preferred_element_type=jnp.float32)
    cnt_ref[...] += jnp.broadcast_to(
        jnp.sum(oh, axis=1, keepdims=True), (NG, D))

    @pl.when(i == GRID - 1)
    def _():
        pooled = sums_ref[...] / jnp.maximum(cnt_ref[...], 1.0)
        o_ref[...] = jnp.dot(pooled, w_ref[...],
                             preferred_element_type=jnp.float32) + bo_ref[...]


_tc3 = pl.pallas_call(
    _tc3_body,
    grid=(GRID,),
    in_specs=[
        pl.BlockSpec((RB, D), lambda i: (i, 0)),
        pl.BlockSpec((1, 1, RB), lambda i: (i, 0, 0)),
        pl.BlockSpec((D, D), lambda i: (0, 0)),
        pl.BlockSpec((1, D), lambda i: (0, 0)),
    ],
    out_specs=pl.BlockSpec((NG, D), lambda i: (0, 0)),
    out_shape=jax.ShapeDtypeStruct((NG, D), jnp.float32),
    scratch_shapes=[
        pltpu.VMEM((NG, D), jnp.float32),
        pltpu.VMEM((NG, D), jnp.float32),
    ],
)


def kernel(x, edge_index, batch, W_in, b_in, W_h, b_h, W_out, b_out):
    ei = edge_index.astype(jnp.int32)
    src2 = ei[0].reshape(NS, NCHS, K)
    batch3 = batch.astype(jnp.int32).reshape(GRID, 1, RB)

    ldst4 = _tc_loc(ei[1].reshape(ER, D)).reshape(NC, NS, NCHS, K)

    zerosD = jnp.zeros((SZR, D), jnp.float32)
    onesK = jnp.ones((K, D), jnp.float32)

    deg2 = _sc_deg(ldst4, onesK, zerosD)
    Ws = jnp.stack([W_in, W_h])
    bs = jnp.stack([b_in.reshape(1, D), b_h.reshape(1, D)])

    def step(h, wb):
        W, b = wb
        g = _tc1(h, W, deg2)
        s = _sc_scatter(src2, ldst4, g, zerosD)
        return _tc_post(s, g, deg2, b), None

    h2, _ = lax.scan(step, x, (Ws, bs))
    return _tc3(h2, batch3, W_out, b_out.reshape(1, D))


# speedup vs baseline: 10.9815x; 1.1861x over previous
import functools

import jax
import jax.numpy as jnp
from jax import lax
from jax.experimental import pallas as pl
from jax.experimental.pallas import tpu as pltpu
from jax.experimental.pallas import tpu_sc as plsc

NN = 10000
NE = 320000
D = 128
NG = 64
NC, NS, L = 2, 16, 16
NW = NC * NS
K = 80
NCHS = (NE // NS) // K
KS = 40
NCHS2 = (NE // NS) // KS
NPH = 2
WCH = NCHS2 // NPH

NH = NN // NC
SWRT = 312
SZR = 24
STAIL = NH - NS * SWRT
ACC_ROWS = 5008

ER = NE // D

RB = 1000
GRID = NN // RB

_mesh = plsc.VectorSubcoreMesh(core_axis_name="c", subcore_axis_name="s")


def _zero_init(zeros_hbm, zbuf_v, sh, s, zr, wrt, tail, tbase):
    pltpu.sync_copy(zeros_hbm, zbuf_v)
    for r in range(wrt // zr):
        pltpu.sync_copy(zbuf_v, sh.at[pl.ds(s * wrt + r * zr, zr)])

    @pl.when(s == 0)
    def _():
        pltpu.sync_copy(zbuf_v.at[pl.ds(0, tail)],
                        sh.at[pl.ds(tbase, tail)])


def _writeout(zbuf_v, sh, out, c, s, zr, wrt, tail, tbase):
    for r in range(wrt // zr):
        rows = pl.ds(s * wrt + r * zr, zr)
        pltpu.sync_copy(sh.at[rows], zbuf_v)
        pltpu.sync_copy(zbuf_v, out.at[c, rows])

    @pl.when(s == 0)
    def _():
        rows = pl.ds(tbase, tail)
        pltpu.sync_copy(sh.at[rows], zbuf_v.at[pl.ds(0, tail)])
        pltpu.sync_copy(zbuf_v.at[pl.ds(0, tail)], out.at[c, rows])


@functools.partial(
    pl.kernel,
    out_type=jax.ShapeDtypeStruct((NC, NH, D), jnp.float32),
    mesh=_mesh,
    scratch_types=[
        pltpu.VMEM((WCH, KS), jnp.int32),
        pltpu.VMEM((WCH, KS), jnp.int32),
        pltpu.VMEM((KS, D), jnp.float32),
        pltpu.VMEM((KS, D), jnp.float32),
        pltpu.VMEM((SZR, D), jnp.float32),
        pltpu.VMEM_SHARED((ACC_ROWS, D), jnp.float32),
        pltpu.SemaphoreType.DMA,
        pltpu.SemaphoreType.DMA,
    ],
)
def _sc_scatter(src2, ldst4, g_hbm, zeros_hbm, out, isrc_v, idst_v, r0_v, r1_v,
                zbuf_v, acc_sh, sem0, sem1):
    c = lax.axis_index("c")
    s = lax.axis_index("s")
    _zero_init(zeros_hbm, zbuf_v, acc_sh, s, SZR, SWRT, STAIL, NS * SWRT)
    plsc.subcore_barrier()

    bufs = (r0_v, r1_v)
    sems = (sem0, sem1)
    for p in range(NPH):
        pltpu.sync_copy(src2.at[s, p], isrc_v)
        pltpu.sync_copy(ldst4.at[c, s, p], idst_v)
        for b in range(2):
            pltpu.async_copy(g_hbm.at[isrc_v.at[b]], bufs[b], sems[b])

        def body(gi, carry):
            for b in range(2):
                j = 2 * gi + b
                pltpu.make_async_copy(g_hbm.at[isrc_v.at[j]], bufs[b],
                                      sems[b]).wait()
                pltpu.sync_copy(bufs[b], acc_sh.at[idst_v.at[j]], add=True)
                pltpu.async_copy(g_hbm.at[isrc_v.at[j + 2]], bufs[b], sems[b])
            return carry

        lax.fori_loop(0, WCH // 2 - 1, body, 0)
        for b in range(2):
            j = WCH - 2 + b
            pltpu.make_async_copy(g_hbm.at[isrc_v.at[j]], bufs[b],
                                  sems[b]).wait()
            pltpu.sync_copy(bufs[b], acc_sh.at[idst_v.at[j]], add=True)
    plsc.subcore_barrier()
    _writeout(zbuf_v, acc_sh, out, c, s, SZR, SWRT, STAIL, NS * SWRT)


@functools.partial(
    pl.kernel,
    out_type=jax.ShapeDtypeStruct((NC, NH, D), jnp.float32),
    mesh=_mesh,
    scratch_types=[
        pltpu.VMEM((NCHS, K), jnp.int32),
        pltpu.VMEM((K, D), jnp.float32),
        pltpu.VMEM((SZR, D), jnp.float32),
        pltpu.VMEM_SHARED((ACC_ROWS, D), jnp.float32),
    ],
)
def _sc_deg(ldst4, ones_hbm, zeros_hbm, out, idst_v, rows_v, zbuf_v, acc_sh):
    c = lax.axis_index("c")
    s = lax.axis_index("s")
    pltpu.sync_copy(ldst4.at[c, s], idst_v)
    pltpu.sync_copy(ones_hbm, rows_v)
    _zero_init(zeros_hbm, zbuf_v, acc_sh, s, SZR, SWRT, STAIL, NS * SWRT)
    plsc.subcore_barrier()

    def body(j, carry):
        pltpu.sync_copy(rows_v, acc_sh.at[idst_v.at[j]], add=True)
        return carry

    lax.fori_loop(0, NCHS, body, 0)
    plsc.subcore_barrier()
    _writeout(zbuf_v, acc_sh, out, c, s, SZR, SWRT, STAIL, NS * SWRT)


def _loc_body(dst_ref, o_ref):
    d = dst_ref[...]
    for c in range(NC):
        t = d - c * NH
        o_ref[c] = jnp.where((t >= 0) & (t < NH), t, NH)


_tc_loc = pl.pallas_call(
    _loc_body,
    out_shape=jax.ShapeDtypeStruct((NC, ER, D), jnp.int32),
)


def _dinv(deg_ref):
    d = deg_ref[0, :, 0] + 1.0
    return lax.rsqrt(d)[:, None]


_HALF_SPEC = pl.BlockSpec((1, RB, D),
                          lambda i: (i // (NH // RB), i % (NH // RB), 0))


def _tc1_body(x_ref, w_ref, deg_ref, o_ref):
    h = jnp.dot(x_ref[...], w_ref[...], preferred_element_type=jnp.float32)
    o_ref[...] = h * _dinv(deg_ref)


_tc1 = pl.pallas_call(
    _tc1_body,
    grid=(GRID,),
    in_specs=[
        pl.BlockSpec((RB, D), lambda i: (i, 0)),
        pl.BlockSpec((D, D), lambda i: (0, 0)),
        _HALF_SPEC,
    ],
    out_specs=pl.BlockSpec((RB, D), lambda i: (i, 0)),
    out_shape=jax.ShapeDtypeStruct((NN, D), jnp.float32),
)


def _post_body(s_ref, g_ref, deg_ref, b_ref, o_ref):
    dinv = _dinv(deg_ref)
    t = (s_ref[0] + g_ref[...]) * dinv + b_ref[...]
    o_ref[...] = jnp.maximum(t, 0.0)


_tc_post = pl.pallas_call(
    _post_body,
    grid=(GRID,),
    in_specs=[
        _HALF_SPEC,
        pl.BlockSpec((RB, D), lambda i: (i, 0)),
        _HALF_SPEC,
        pl.BlockSpec((1, D), lambda i: (0, 0)),
    ],
    out_specs=pl.BlockSpec((RB, D), lambda i: (i, 0)),
    out_shape=jax.ShapeDtypeStruct((NN, D), jnp.float32),
)


def _tc3_body(h_ref, batch_ref, w_ref, bo_ref, o_ref, sums_ref, cnt_ref):
    i = pl.program_id(0)

    @pl.when(i == 0)
    def _():
        sums_ref[...] = jnp.zeros_like(sums_ref)
        cnt_ref[...] = jnp.zeros_like(cnt_ref)

    h = h_ref[...]
    bt = batch_ref[0, 0, :]
    oh = (lax.broadcasted_iota(jnp.int32, (NG, RB), 0) == bt[None, :]).astype(
        jnp.float32)
    sums_ref[...] += jnp.dot(oh, h, preferred_element_type=jnp.float32)
    cnt_ref[...] += jnp.broadcast_to(
        jnp.sum(oh, axis=1, keepdims=True), (NG, D))

    @pl.when(i == GRID - 1)
    def _():
        pooled = sums_ref[...] / jnp.maximum(cnt_ref[...], 1.0)
        o_ref[...] = jnp.dot(pooled, w_ref[...],
                             preferred_element_type=jnp.float32) + bo_ref[...]


_tc3 = pl.pallas_call(
    _tc3_body,
    grid=(GRID,),
    in_specs=[
        pl.BlockSpec((RB, D), lambda i: (i, 0)),
        pl.BlockSpec((1, 1, RB), lambda i: (i, 0, 0)),
        pl.BlockSpec((D, D), lambda i: (0, 0)),
        pl.BlockSpec((1, D), lambda i: (0, 0)),
    ],
    out_specs=pl.BlockSpec((NG, D), lambda i: (0, 0)),
    out_shape=jax.ShapeDtypeStruct((NG, D), jnp.float32),
    scratch_shapes=[
        pltpu.VMEM((NG, D), jnp.float32),
        pltpu.VMEM((NG, D), jnp.float32),
    ],
)


def kernel(x, edge_index, batch, W_in, b_in, W_h, b_h, W_out, b_out):
    ei = edge_index.astype(jnp.int32)
    src2 = ei[0].reshape(NS, NPH, WCH, KS)
    batch3 = batch.astype(jnp.int32).reshape(GRID, 1, RB)

    ldst = _tc_loc(ei[1].reshape(ER, D))
    ldst4 = ldst.reshape(NC, NS, NCHS, K)
    ldst4s = ldst.reshape(NC, NS, NPH, WCH, KS)

    zerosD = jnp.zeros((SZR, D), jnp.float32)
    onesK = jnp.ones((K, D), jnp.float32)

    deg2 = _sc_deg(ldst4, onesK, zerosD)
    Ws = jnp.stack([W_in, W_h])
    bs = jnp.stack([b_in.reshape(1, D), b_h.reshape(1, D)])

    def step(h, wb):
        W, b = wb
        g = _tc1(h, W, deg2)
        s = _sc_scatter(src2, ldst4s, g, zerosD)
        return _tc_post(s, g, deg2, b), None

    h2, _ = lax.scan(step, x, (Ws, bs))
    return _tc3(h2, batch3, W_out, b_out.reshape(1, D))


# speedup vs baseline: 12.3613x; 1.1256x over previous
import functools

import jax
import jax.numpy as jnp
from jax import lax
from jax.experimental import pallas as pl
from jax.experimental.pallas import tpu as pltpu
from jax.experimental.pallas import tpu_sc as plsc

NN = 10000
NE = 320000
D = 128
NG = 64
NC, NS, L = 2, 16, 16
NW = NC * NS
K = 80
NCHS = (NE // NS) // K
KS = 80
NCHS2 = (NE // NS) // KS
NPH = 2
WCH = NCHS2 // NPH

NH = NN // NC
SWRT = 312
SZR = 24
STAIL = NH - NS * SWRT
ACC_ROWS = 5008

ER = NE // D

RB = 1000
GRID = NN // RB

_mesh = plsc.VectorSubcoreMesh(core_axis_name="c", subcore_axis_name="s")


def _zero_init(zeros_hbm, zbuf_v, sh, s, zr, wrt, tail, tbase):
    pltpu.sync_copy(zeros_hbm, zbuf_v)
    for r in range(wrt // zr):
        pltpu.sync_copy(zbuf_v, sh.at[pl.ds(s * wrt + r * zr, zr)])

    @pl.when(s == 0)
    def _():
        pltpu.sync_copy(zbuf_v.at[pl.ds(0, tail)],
                        sh.at[pl.ds(tbase, tail)])


def _writeout(zbuf_v, sh, out, c, s, zr, wrt, tail, tbase):
    for r in range(wrt // zr):
        rows = pl.ds(s * wrt + r * zr, zr)
        pltpu.sync_copy(sh.at[rows], zbuf_v)
        pltpu.sync_copy(zbuf_v, out.at[c, rows])

    @pl.when(s == 0)
    def _():
        rows = pl.ds(tbase, tail)
        pltpu.sync_copy(sh.at[rows], zbuf_v.at[pl.ds(0, tail)])
        pltpu.sync_copy(zbuf_v.at[pl.ds(0, tail)], out.at[c, rows])


@functools.partial(
    pl.kernel,
    out_type=jax.ShapeDtypeStruct((NC, NH, D), jnp.float32),
    mesh=_mesh,
    scratch_types=[
        pltpu.VMEM((WCH, KS), jnp.int32),
        pltpu.VMEM((WCH, KS), jnp.int32),
        pltpu.VMEM((KS, D), jnp.float32),
        pltpu.VMEM((KS, D), jnp.float32),
        pltpu.VMEM((SZR, D), jnp.float32),
        pltpu.VMEM_SHARED((ACC_ROWS, D), jnp.float32),
        pltpu.SemaphoreType.DMA,
        pltpu.SemaphoreType.DMA,
    ],
)
def _sc_scatter(src2, ldst4, g_hbm, zeros_hbm, out, isrc_v, idst_v, r0_v, r1_v,
                zbuf_v, acc_sh, sem0, sem1):
    c = lax.axis_index("c")
    s = lax.axis_index("s")
    _zero_init(zeros_hbm, zbuf_v, acc_sh, s, SZR, SWRT, STAIL, NS * SWRT)
    plsc.subcore_barrier()

    bufs = (r0_v, r1_v)
    sems = (sem0, sem1)
    for p in range(NPH):
        pltpu.sync_copy(src2.at[s, p], isrc_v)
        pltpu.sync_copy(ldst4.at[c, s, p], idst_v)
        for b in range(2):
            pltpu.async_copy(g_hbm.at[isrc_v.at[b]], bufs[b], sems[b])

        def body(gi, carry):
            for b in range(2):
                j = 2 * gi + b
                pltpu.make_async_copy(g_hbm.at[isrc_v.at[j]], bufs[b],
                                      sems[b]).wait()
                pltpu.sync_copy(bufs[b], acc_sh.at[idst_v.at[j]], add=True)
                pltpu.async_copy(g_hbm.at[isrc_v.at[j + 2]], bufs[b], sems[b])
            return carry

        niter = (WCH - 3) // 2 if WCH % 2 else WCH // 2 - 1
        lax.fori_loop(0, niter, body, 0)
        tail = [(2 * niter + t, t % 2) for t in range(WCH - 2 * niter)]
        for t, (j, b) in enumerate(tail):
            pltpu.make_async_copy(g_hbm.at[isrc_v.at[j]], bufs[b],
                                  sems[b]).wait()
            pltpu.sync_copy(bufs[b], acc_sh.at[idst_v.at[j]], add=True)
            if t + 2 < len(tail):
                pltpu.async_copy(g_hbm.at[isrc_v.at[j + 2]], bufs[b], sems[b])
    plsc.subcore_barrier()
    _writeout(zbuf_v, acc_sh, out, c, s, SZR, SWRT, STAIL, NS * SWRT)


@functools.partial(
    pl.kernel,
    out_type=jax.ShapeDtypeStruct((NC, NH, D), jnp.float32),
    mesh=_mesh,
    scratch_types=[
        pltpu.VMEM((NCHS, K), jnp.int32),
        pltpu.VMEM((K, D), jnp.float32),
        pltpu.VMEM((SZR, D), jnp.float32),
        pltpu.VMEM_SHARED((ACC_ROWS, D), jnp.float32),
    ],
)
def _sc_deg(ldst4, ones_hbm, zeros_hbm, out, idst_v, rows_v, zbuf_v, acc_sh):
    c = lax.axis_index("c")
    s = lax.axis_index("s")
    pltpu.sync_copy(ldst4.at[c, s], idst_v)
    pltpu.sync_copy(ones_hbm, rows_v)
    _zero_init(zeros_hbm, zbuf_v, acc_sh, s, SZR, SWRT, STAIL, NS * SWRT)
    plsc.subcore_barrier()

    def body(j, carry):
        pltpu.sync_copy(rows_v, acc_sh.at[idst_v.at[j]], add=True)
        return carry

    lax.fori_loop(0, NCHS, body, 0)
    plsc.subcore_barrier()
    _writeout(zbuf_v, acc_sh, out, c, s, SZR, SWRT, STAIL, NS * SWRT)


def _loc_body(dst_ref, o_ref):
    d = dst_ref[...]
    for c in range(NC):
        t = d - c * NH
        o_ref[c] = jnp.where((t >= 0) & (t < NH), t, NH)


_tc_loc = pl.pallas_call(
    _loc_body,
    out_shape=jax.ShapeDtypeStruct((NC, ER, D), jnp.int32),
)


def _dinv(deg_ref):
    d = deg_ref[0, :, 0] + 1.0
    return lax.rsqrt(d)[:, None]


_HALF_SPEC = pl.BlockSpec((1, RB, D),
                          lambda i: (i // (NH // RB), i % (NH // RB), 0))


def _tc1_body(x_ref, w_ref, deg_ref, o_ref):
    h = jnp.dot(x_ref[...], w_ref[...], preferred_element_type=jnp.float32)
    o_ref[...] = h * _dinv(deg_ref)


_tc1 = pl.pallas_call(
    _tc1_body,
    grid=(GRID,),
    in_specs=[
        pl.BlockSpec((RB, D), lambda i: (i, 0)),
        pl.BlockSpec((D, D), lambda i: (0, 0)),
        _HALF_SPEC,
    ],
    out_specs=pl.BlockSpec((RB, D), lambda i: (i, 0)),
    out_shape=jax.ShapeDtypeStruct((NN, D), jnp.float32),
)


def _post_body(s_ref, g_ref, deg_ref, b_ref, o_ref):
    dinv = _dinv(deg_ref)
    t = (s_ref[0] + g_ref[...]) * dinv + b_ref[...]
    o_ref[...] = jnp.maximum(t, 0.0)


_tc_post = pl.pallas_call(
    _post_body,
    grid=(GRID,),
    in_specs=[
        _HALF_SPEC,
        pl.BlockSpec((RB, D), lambda i: (i, 0)),
        _HALF_SPEC,
        pl.BlockSpec((1, D), lambda i: (0, 0)),
    ],
    out_specs=pl.BlockSpec((RB, D), lambda i: (i, 0)),
    out_shape=jax.ShapeDtypeStruct((NN, D), jnp.float32),
)


def _tc3_body(h_ref, batch_ref, w_ref, bo_ref, o_ref, sums_ref, cnt_ref):
    i = pl.program_id(0)

    @pl.when(i == 0)
    def _():
        sums_ref[...] = jnp.zeros_like(sums_ref)
        cnt_ref[...] = jnp.zeros_like(cnt_ref)

    h = h_ref[...]
    bt = batch_ref[0, 0, :]
    oh = (lax.broadcasted_iota(jnp.int32, (NG, RB), 0) == bt[None, :]).astype(
        jnp.float32)
    sums_ref[...] += jnp.dot(oh, h, preferred_element_type=jnp.float32)
    cnt_ref[...] += jnp.broadcast_to(
        jnp.sum(oh, axis=1, keepdims=True), (NG, D))

    @pl.when(i == GRID - 1)
    def _():
        pooled = sums_ref[...] / jnp.maximum(cnt_ref[...], 1.0)
        o_ref[...] = jnp.dot(pooled, w_ref[...],
                             preferred_element_type=jnp.float32) + bo_ref[...]


_tc3 = pl.pallas_call(
    _tc3_body,
    grid=(GRID,),
    in_specs=[
        pl.BlockSpec((RB, D), lambda i: (i, 0)),
        pl.BlockSpec((1, 1, RB), lambda i: (i, 0, 0)),
        pl.BlockSpec((D, D), lambda i: (0, 0)),
        pl.BlockSpec((1, D), lambda i: (0, 0)),
    ],
    out_specs=pl.BlockSpec((NG, D), lambda i: (0, 0)),
    out_shape=jax.ShapeDtypeStruct((NG, D), jnp.float32),
    scratch_shapes=[
        pltpu.VMEM((NG, D), jnp.float32),
        pltpu.VMEM((NG, D), jnp.float32),
    ],
)


def kernel(x, edge_index, batch, W_in, b_in, W_h, b_h, W_out, b_out):
    ei = edge_index.astype(jnp.int32)
    src2 = ei[0].reshape(NS, NPH, WCH, KS)
    batch3 = batch.astype(jnp.int32).reshape(GRID, 1, RB)

    ldst = _tc_loc(ei[1].reshape(ER, D))
    ldst4 = ldst.reshape(NC, NS, NCHS, K)
    ldst4s = ldst.reshape(NC, NS, NPH, WCH, KS)

    zerosD = jnp.zeros((SZR, D), jnp.float32)
    onesK = jnp.ones((K, D), jnp.float32)

    deg2 = _sc_deg(ldst4, onesK, zerosD)
    Ws = jnp.stack([W_in, W_h])
    bs = jnp.stack([b_in.reshape(1, D), b_h.reshape(1, D)])

    def step(h, wb):
        W, b = wb
        g = _tc1(h, W, deg2)
        s = _sc_scatter(src2, ldst4s, g, zerosD)
        return _tc_post(s, g, deg2, b), None

    h2, _ = lax.scan(step, x, (Ws, bs))
    return _tc3(h2, batch3, W_out, b_out.reshape(1, D))
